# baseline (speedup 1.0000x reference)
import functools

import jax
import jax.numpy as jnp
from jax import lax
from jax.experimental import pallas as pl
from jax.experimental.pallas import tpu as pltpu, tpu_sc as plsc

N = 10000
E = 320000
D = 128
H = 64
OUT = 64
G = 64

NC, NS = 2, 16
NW = NC * NS
CW = 16
AW = H + CW

NSUB, SUBW = 1, 250
CHUNK = NSUB * SUBW
EPT = E // NW
NCHUNK = EPT // CHUNK
DEPTH = 3
FRT = N // NS

RT = 1000
NBLK = N // RT


def _edge_agg_body(w, *refs):
    if w == AW:
        (p_hbm, er, z, out_d, out_c, sidx, didx, rows, acc, gsem, ssem) = refs
    else:
        (p_hbm, er, z, out_d, sidx, didx, rows, acc, gsem, ssem) = refs
        out_c = None

    cid = lax.axis_index("c")
    sid = lax.axis_index("s")
    wid = sid * NC + cid

    fsl = pl.ds(sid * FRT, FRT)
    pltpu.sync_copy(er.at[0, wid], sidx)
    pltpu.sync_copy(er.at[1, wid], didx)

    def fire_gathers(s, b):
        for j in range(NSUB):
            pltpu.make_async_copy(p_hbm.at[sidx.at[s, j]], rows.at[b, j],
                                  gsem).start()

    def wait_gathers(b):
        for j in range(NSUB):
            pltpu.make_async_copy(p_hbm.at[sidx.at[0, j]], rows.at[b, j],
                                  gsem).wait()

    def fire_scatters(s, b):
        for j in range(NSUB):
            pltpu.make_async_copy(rows.at[b, j, :, pl.ds(0, w)],
                                  acc.at[didx.at[s, j]],
                                  ssem).start(add=True)

    def drain_scatters(s, b):
        for j in range(NSUB):
            pltpu.make_async_copy(rows.at[b, j, :, pl.ds(0, w)],
                                  acc.at[didx.at[s, j]],
                                  ssem).wait()

    for k in range(DEPTH - 1):
        fire_gathers(k, k)
    pltpu.sync_copy(z, acc.at[fsl])
    plsc.subcore_barrier()

    def chunk(s, carry):
        b = lax.rem(s, DEPTH)
        pf = s + DEPTH - 1

        @pl.when(pf < NCHUNK)
        def _():
            @pl.when(s >= 1)
            def _():
                drain_scatters(s - 1, lax.rem(s - 1, DEPTH))
            fire_gathers(pf, lax.rem(pf, DEPTH))

        wait_gathers(b)
        fire_scatters(s, b)
        return carry

    lax.fori_loop(0, NCHUNK, chunk, 0)
    for s in range(NCHUNK - DEPTH, NCHUNK):
        drain_scatters(s, s % DEPTH)
    plsc.subcore_barrier()

    if w == AW:
        pltpu.sync_copy(acc.at[fsl, pl.ds(0, H)],
                        out_d.at[fsl, pl.ds(cid * H, H)])
        pltpu.sync_copy(acc.at[fsl, pl.ds(H, CW)],
                        out_c.at[fsl, pl.ds(cid * CW, CW)])
    else:
        pltpu.sync_copy(acc.at[fsl], out_d.at[fsl, pl.ds(cid * H, H)])


def _make_edge_agg(w):
    mesh = plsc.VectorSubcoreMesh(core_axis_name="c", subcore_axis_name="s",
                                  num_cores=NC, num_subcores=NS)
    out_type = [jax.ShapeDtypeStruct((N, 2 * H), jnp.float32)]
    if w == AW:
        out_type.append(jax.ShapeDtypeStruct((N, 2 * H), jnp.float32))
    return pl.kernel(
        functools.partial(_edge_agg_body, w),
        out_type=out_type,
        mesh=mesh,
        scratch_types=[
            pltpu.VMEM((NCHUNK, NSUB, SUBW), jnp.int32),
            pltpu.VMEM((NCHUNK, NSUB, SUBW), jnp.int32),
            pltpu.VMEM((DEPTH, NSUB, SUBW, w), jnp.float32),
            pltpu.VMEM_SHARED((N, w), jnp.float32),
            pltpu.SemaphoreType.DMA,
            pltpu.SemaphoreType.DMA,
        ],
        compiler_params=pltpu.CompilerParams(use_tc_tiling_on_sc=False),
        name=f"edge_agg_w{w}",
    )


def _pre_body(x_ref, w1l_ref, w1r_ref, b1_ref, p1_ref, r1_ref):
    xb = x_ref[...]
    dn = (((1,), (1,)), ((), ()))
    p1 = lax.dot_general(xb, w1l_ref[...], dn,
                         preferred_element_type=jnp.float32)
    p1_ref[...] = jnp.concatenate([p1, jnp.ones((RT, CW), jnp.float32)],
                                  axis=1)
    r1_ref[...] = lax.dot_general(xb, w1r_ref[...], dn,
                                  preferred_element_type=jnp.float32) + b1_ref[0:1, :]


def _mid_body(d_ref, c_ref, r1_ref, w2l_ref, w2r_ref, b2_ref,
              p2_ref, q2_ref):
    d = d_ref[...]
    c = c_ref[...]
    inv = 1.0 / jnp.maximum(c[:, 0:1] + c[:, CW:CW + 1], 1.0)
    h = jnp.maximum((d[:, :H] + d[:, H:]) * inv + r1_ref[...], 0.0)
    dn = (((1,), (1,)), ((), ()))
    p2_ref[...] = lax.dot_general(h, w2l_ref[...], dn,
                                  preferred_element_type=jnp.float32)
    q2 = lax.dot_general(h, w2r_ref[...], dn,
                         preferred_element_type=jnp.float32) + b2_ref[0:1, :]
    q2_ref[...] = jnp.concatenate([q2, jnp.broadcast_to(inv, (RT, CW))],
                                  axis=1)


def _pool_body(d_ref, q2_ref, b_ref, out_ref):
    i = pl.program_id(0)
    d = d_ref[...]
    q2a = q2_ref[...]
    h2 = (d[:, :H] + d[:, H:]) * q2a[:, H:H + 1] + q2a[:, :H]
    bidx = b_ref[0, 0, :]
    onehot = (bidx[:, None] == lax.broadcasted_iota(jnp.int32, (1, G), 1)
              ).astype(jnp.float32)
    h2a = jnp.concatenate([h2, jnp.ones((RT, 64), jnp.float32)], axis=1)
    contrib = lax.dot_general(onehot, h2a, (((0,), (0,)), ((), ())),
                              preferred_element_type=jnp.float32)

    @pl.when(i == 0)
    def _():
        out_ref[...] = contrib

    @pl.when(i > 0)
    def _():
        out_ref[...] += contrib

    @pl.when(i == NBLK - 1)
    def _():
        o = out_ref[...]
        out_ref[...] = o / jnp.maximum(o[:, 64:65], 1.0)


_full = lambda i: (0, 0)


def kernel(x, edge_index, batch_index, W1l, W1r, b1, W2l, W2r, b2):
    f32 = jnp.float32
    er = edge_index.reshape(2, NW, NCHUNK, NSUB, SUBW)
    bidx = batch_index.reshape(NBLK, 1, RT)
    z80 = jnp.zeros((FRT, AW), f32)
    z64 = jnp.zeros((FRT, H), f32)
    b1t = jnp.broadcast_to(b1, (8, H))
    b2t = jnp.broadcast_to(b2, (8, OUT))

    p1, r1 = pl.pallas_call(
        _pre_body,
        grid=(NBLK,),
        in_specs=[
            pl.BlockSpec((RT, D), lambda i: (i, 0)),
            pl.BlockSpec((H, D), _full),
            pl.BlockSpec((H, D), _full),
            pl.BlockSpec((8, H), _full),
        ],
        out_specs=[pl.BlockSpec((RT, AW), lambda i: (i, 0)),
                   pl.BlockSpec((RT, H), lambda i: (i, 0))],
        out_shape=[jax.ShapeDtypeStruct((N, AW), f32),
                   jax.ShapeDtypeStruct((N, H), f32)],
    )(x, W1l, W1r, b1t)

    d1, c1 = _make_edge_agg(AW)(p1, er, z80)

    row128 = pl.BlockSpec((RT, 2 * H), lambda i: (i, 0))
    p2, q2 = pl.pallas_call(
        _mid_body,
        grid=(NBLK,),
        in_specs=[
            row128, row128,
            pl.BlockSpec((RT, H), lambda i: (i, 0)),
            pl.BlockSpec((OUT, H), _full),
            pl.BlockSpec((OUT, H), _full),
            pl.BlockSpec((8, OUT), _full),
        ],
        out_specs=[pl.BlockSpec((RT, OUT), lambda i: (i, 0)),
                   pl.BlockSpec((RT, AW), lambda i: (i, 0))],
        out_shape=[jax.ShapeDtypeStruct((N, OUT), f32),
                   jax.ShapeDtypeStruct((N, AW), f32)],
    )(d1, c1, r1, W2l, W2r, b2t)

    d2, = _make_edge_agg(H)(p2, er, z64)

    pooled = pl.pallas_call(
        _pool_body,
        grid=(NBLK,),
        in_specs=[
            row128,
            pl.BlockSpec((RT, AW), lambda i: (i, 0)),
            pl.BlockSpec((1, 1, RT), lambda i: (i, 0, 0)),
        ],
        out_specs=pl.BlockSpec((G, 128), _full),
        out_shape=jax.ShapeDtypeStruct((G, 128), f32),
    )(d2, q2, bidx)

    return pooled[:, :OUT]


# speedup vs baseline: 1.0007x; 1.0007x over previous
import functools

import jax
import jax.numpy as jnp
from jax import lax
from jax.experimental import pallas as pl
from jax.experimental.pallas import tpu as pltpu, tpu_sc as plsc

N = 10000
E = 320000
D = 128
H = 64
OUT = 64
G = 64

NC, NS = 2, 16
NW = NC * NS
CW = 16
AW = H + CW

NSUB, SUBW = 1, 250
CHUNK = NSUB * SUBW
EPT = E // NW
NCHUNK = EPT // CHUNK
DEPTH = 3
FRT = N // NS

RT = 1000
NBLK = N // RT


def _edge_agg_body(w, *refs):
    if w == AW:
        (p_hbm, er, z, out_d, out_c, sidx, didx, rows, acc, gsem, ssem) = refs
    else:
        (p_hbm, er, z, out_d, sidx, didx, rows, acc, gsem, ssem) = refs
        out_c = None

    cid = lax.axis_index("c")
    sid = lax.axis_index("s")
    wid = sid * NC + cid

    fsl = pl.ds(sid * FRT, FRT)
    pltpu.sync_copy(er.at[0, wid], sidx)
    pltpu.sync_copy(er.at[1, wid], didx)

    def fire_gathers(s, b):
        for j in range(NSUB):
            pltpu.make_async_copy(p_hbm.at[sidx.at[s, j]], rows.at[b, j],
                                  gsem).start()

    def wait_gathers(b):
        for j in range(NSUB):
            pltpu.make_async_copy(p_hbm.at[sidx.at[0, j]], rows.at[b, j],
                                  gsem).wait()

    def fire_scatters(s, b):
        for j in range(NSUB):
            pltpu.make_async_copy(rows.at[b, j, :, pl.ds(0, w)],
                                  acc.at[didx.at[s, j]],
                                  ssem).start(add=True)

    def drain_scatters(s, b):
        for j in range(NSUB):
            pltpu.make_async_copy(rows.at[b, j, :, pl.ds(0, w)],
                                  acc.at[didx.at[s, j]],
                                  ssem).wait()

    for k in range(DEPTH - 1):
        fire_gathers(k, k)
    pltpu.sync_copy(z, acc.at[fsl])
    plsc.subcore_barrier()

    def chunk(s, carry):
        b = lax.rem(s, DEPTH)
        pf = s + DEPTH - 1

        @pl.when(pf < NCHUNK)
        def _():
            @pl.when(s >= 1)
            def _():
                drain_scatters(s - 1, lax.rem(s - 1, DEPTH))
            fire_gathers(pf, lax.rem(pf, DEPTH))

        wait_gathers(b)
        fire_scatters(s, b)
        return carry

    lax.fori_loop(0, NCHUNK, chunk, 0)
    for s in range(NCHUNK - DEPTH, NCHUNK):
        drain_scatters(s, s % DEPTH)
    plsc.subcore_barrier()

    if w == AW:
        pltpu.sync_copy(acc.at[fsl, pl.ds(0, H)],
                        out_d.at[fsl, pl.ds(cid * H, H)])
        pltpu.sync_copy(acc.at[fsl, pl.ds(H, CW)],
                        out_c.at[fsl, pl.ds(cid * CW, CW)])
    else:
        pltpu.sync_copy(acc.at[fsl], out_d.at[fsl, pl.ds(cid * H, H)])


def _make_edge_agg(w):
    mesh = plsc.VectorSubcoreMesh(core_axis_name="c", subcore_axis_name="s",
                                  num_cores=NC, num_subcores=NS)
    out_type = [jax.ShapeDtypeStruct((N, 2 * H), jnp.float32)]
    if w == AW:
        out_type.append(jax.ShapeDtypeStruct((N, 2 * H), jnp.float32))
    return pl.kernel(
        functools.partial(_edge_agg_body, w),
        out_type=out_type,
        mesh=mesh,
        scratch_types=[
            pltpu.VMEM((NCHUNK, NSUB, SUBW), jnp.int32),
            pltpu.VMEM((NCHUNK, NSUB, SUBW), jnp.int32),
            pltpu.VMEM((DEPTH, NSUB, SUBW, w), jnp.float32),
            pltpu.VMEM_SHARED((N, w), jnp.float32),
            pltpu.SemaphoreType.DMA,
            pltpu.SemaphoreType.DMA,
        ],
        compiler_params=pltpu.CompilerParams(use_tc_tiling_on_sc=False),
        name=f"edge_agg_w{w}",
    )


def _pre_p_body(x_ref, w1l_ref, p1_ref):
    dn = (((1,), (1,)), ((), ()))
    p1 = lax.dot_general(x_ref[...], w1l_ref[...], dn,
                         preferred_element_type=jnp.float32)
    p1_ref[...] = jnp.concatenate([p1, jnp.ones((RT, CW), jnp.float32)],
                                  axis=1)


def _pre_r_body(x_ref, w1r_ref, b1_ref, r1_ref):
    dn = (((1,), (1,)), ((), ()))
    r1_ref[...] = lax.dot_general(x_ref[...], w1r_ref[...], dn,
                                  preferred_element_type=jnp.float32) + b1_ref[0:1, :]


def _mid_body(d_ref, c_ref, r1_ref, w2l_ref, w2r_ref, b2_ref,
              p2_ref, q2_ref):
    d = d_ref[...]
    c = c_ref[...]
    inv = 1.0 / jnp.maximum(c[:, 0:1] + c[:, CW:CW + 1], 1.0)
    h = jnp.maximum((d[:, :H] + d[:, H:]) * inv + r1_ref[...], 0.0)
    dn = (((1,), (1,)), ((), ()))
    p2_ref[...] = lax.dot_general(h, w2l_ref[...], dn,
                                  preferred_element_type=jnp.float32)
    q2 = lax.dot_general(h, w2r_ref[...], dn,
                         preferred_element_type=jnp.float32) + b2_ref[0:1, :]
    q2_ref[...] = jnp.concatenate([q2, jnp.broadcast_to(inv, (RT, CW))],
                                  axis=1)


def _pool_body(d_ref, q2_ref, b_ref, out_ref):
    i = pl.program_id(0)
    d = d_ref[...]
    q2a = q2_ref[...]
    h2 = (d[:, :H] + d[:, H:]) * q2a[:, H:H + 1] + q2a[:, :H]
    bidx = b_ref[0, 0, :]
    onehot = (bidx[:, None] == lax.broadcasted_iota(jnp.int32, (1, G), 1)
              ).astype(jnp.float32)
    h2a = jnp.concatenate([h2, jnp.ones((RT, 64), jnp.float32)], axis=1)
    contrib = lax.dot_general(onehot, h2a, (((0,), (0,)), ((), ())),
                              preferred_element_type=jnp.float32)

    @pl.when(i == 0)
    def _():
        out_ref[...] = contrib

    @pl.when(i > 0)
    def _():
        out_ref[...] += contrib

    @pl.when(i == NBLK - 1)
    def _():
        o = out_ref[...]
        out_ref[...] = o / jnp.maximum(o[:, 64:65], 1.0)


_full = lambda i: (0, 0)


def kernel(x, edge_index, batch_index, W1l, W1r, b1, W2l, W2r, b2):
    f32 = jnp.float32
    er = edge_index.reshape(2, NW, NCHUNK, NSUB, SUBW)
    bidx = batch_index.reshape(NBLK, 1, RT)
    z80 = jnp.zeros((FRT, AW), f32)
    z64 = jnp.zeros((FRT, H), f32)
    b1t = jnp.broadcast_to(b1, (8, H))
    b2t = jnp.broadcast_to(b2, (8, OUT))

    p1 = pl.pallas_call(
        _pre_p_body,
        grid=(NBLK,),
        in_specs=[
            pl.BlockSpec((RT, D), lambda i: (i, 0)),
            pl.BlockSpec((H, D), _full),
        ],
        out_specs=pl.BlockSpec((RT, AW), lambda i: (i, 0)),
        out_shape=jax.ShapeDtypeStruct((N, AW), f32),
    )(x, W1l)
    r1 = pl.pallas_call(
        _pre_r_body,
        grid=(NBLK,),
        in_specs=[
            pl.BlockSpec((RT, D), lambda i: (i, 0)),
            pl.BlockSpec((H, D), _full),
            pl.BlockSpec((8, H), _full),
        ],
        out_specs=pl.BlockSpec((RT, H), lambda i: (i, 0)),
        out_shape=jax.ShapeDtypeStruct((N, H), f32),
    )(x, W1r, b1t)

    d1, c1 = _make_edge_agg(AW)(p1, er, z80)

    row128 = pl.BlockSpec((RT, 2 * H), lambda i: (i, 0))
    p2, q2 = pl.pallas_call(
        _mid_body,
        grid=(NBLK,),
        in_specs=[
            row128, row128,
            pl.BlockSpec((RT, H), lambda i: (i, 0)),
            pl.BlockSpec((OUT, H), _full),
            pl.BlockSpec((OUT, H), _full),
            pl.BlockSpec((8, OUT), _full),
        ],
        out_specs=[pl.BlockSpec((RT, OUT), lambda i: (i, 0)),
                   pl.BlockSpec((RT, AW), lambda i: (i, 0))],
        out_shape=[jax.ShapeDtypeStruct((N, OUT), f32),
                   jax.ShapeDtypeStruct((N, AW), f32)],
    )(d1, c1, r1, W2l, W2r, b2t)

    d2, = _make_edge_agg(H)(p2, er, z64)

    pooled = pl.pallas_call(
        _pool_body,
        grid=(NBLK,),
        in_specs=[
            row128,
            pl.BlockSpec((RT, AW), lambda i: (i, 0)),
            pl.BlockSpec((1, 1, RT), lambda i: (i, 0, 0)),
        ],
        out_specs=pl.BlockSpec((G, 128), _full),
        out_shape=jax.ShapeDtypeStruct((G, 128), f32),
    )(d2, q2, bidx)

    return pooled[:, :OUT]
